# Initial kernel scaffold; baseline (speedup 1.0000x reference)
import jax
import jax.numpy as jnp
from jax.experimental import pallas as pl


def kernel(x, pos, edge_index, mW1, mb1, mW2, mb2, uW1, ub1, uW2, ub2):
    raise NotImplementedError("write your pallas kernel here")


# speedup vs baseline: 3.7559x; 3.7559x over previous
import functools

import jax
import jax.numpy as jnp
from jax import lax
from jax.experimental import pallas as pl
from jax.experimental.pallas import tpu as pltpu
from jax.experimental.pallas import tpu_sc as plsc

NC = 2
NS = 16
NW = NC * NS
K = 400


def _sigmoid(v):
    return 1.0 / (1.0 + jnp.exp(-v))


def _silu(v):
    return v * _sigmoid(v)


def _precompute_body(x_ref, wst_ref, wrt_ref, xs_ref, xr_ref):
    xb = x_ref[...]
    xs_ref[...] = jnp.dot(xb, wst_ref[...], preferred_element_type=jnp.float32)
    xr_ref[...] = jnp.dot(xb, wrt_ref[...], preferred_element_type=jnp.float32)


def _precompute(x, wst, wrt, nb):
    n, d = x.shape
    grid = (n // nb,)
    return pl.pallas_call(
        _precompute_body,
        grid=grid,
        in_specs=[
            pl.BlockSpec((nb, d), lambda i: (i, 0)),
            pl.BlockSpec((d, d), lambda i: (0, 0)),
            pl.BlockSpec((d, d), lambda i: (0, 0)),
        ],
        out_specs=[
            pl.BlockSpec((nb, d), lambda i: (i, 0)),
            pl.BlockSpec((nb, d), lambda i: (i, 0)),
        ],
        out_shape=[
            jax.ShapeDtypeStruct((n, d), jnp.float32),
            jax.ShapeDtypeStruct((n, d), jnp.float32),
        ],
    )(x, wst, wrt)


def _make_gather(n, d, e, dp):
    ep = e // NW
    nchunk = ep // K
    mesh = plsc.VectorSubcoreMesh(core_axis_name="c", subcore_axis_name="s")

    @functools.partial(
        pl.kernel,
        mesh=mesh,
        out_type=[
            jax.ShapeDtypeStruct((e, d), jnp.float32),
            jax.ShapeDtypeStruct((e, d), jnp.float32),
            jax.ShapeDtypeStruct((e, dp), jnp.float32),
            jax.ShapeDtypeStruct((e, dp), jnp.float32),
        ],
        scratch_types=[
            pltpu.VMEM((K,), jnp.int32),
            pltpu.VMEM((K,), jnp.int32),
            pltpu.VMEM((K, d), jnp.float32),
            pltpu.VMEM((K, dp), jnp.float32),
            pltpu.SemaphoreType.DMA,
            pltpu.SemaphoreType.DMA,
        ],
        compiler_params=pltpu.CompilerParams(use_tc_tiling_on_sc=False),
    )
    def gather_kernel(xs_hbm, xr_hbm, send_hbm, rec_hbm, pp_hbm,
                      gs_hbm, gr_hbm, ps_hbm, pr_hbm,
                      sidx_v, ridx_v, gbuf, pbuf, sem, psem):
        c = lax.axis_index("c")
        s = lax.axis_index("s")
        wid = s * NC + c

        def chunk(i, carry):
            base = wid * ep + i * K
            pltpu.sync_copy(send_hbm.at[pl.ds(base, K)], sidx_v)
            pltpu.sync_copy(rec_hbm.at[pl.ds(base, K)], ridx_v)
            gcp = pltpu.async_copy(xs_hbm.at[sidx_v], gbuf, sem)
            pcp = pltpu.async_copy(pp_hbm.at[sidx_v], pbuf, psem)
            gcp.wait()
            pltpu.sync_copy(gbuf, gs_hbm.at[pl.ds(base, K)])
            pcp.wait()
            pltpu.sync_copy(pbuf, ps_hbm.at[pl.ds(base, K)])
            gcp = pltpu.async_copy(xr_hbm.at[ridx_v], gbuf, sem)
            pcp = pltpu.async_copy(pp_hbm.at[ridx_v], pbuf, psem)
            gcp.wait()
            pltpu.sync_copy(gbuf, gr_hbm.at[pl.ds(base, K)])
            pcp.wait()
            pltpu.sync_copy(pbuf, pr_hbm.at[pl.ds(base, K)])
            return carry

        lax.fori_loop(0, nchunk, chunk, 0)

    return gather_kernel


def _edge_mlp_body(gs_ref, gr_ref, ps_ref, pr_ref, wd_ref, mb1_ref, w2t_ref,
                   mb2_ref, msg_ref):
    diff = ps_ref[...] - pr_ref[...]
    dist = jnp.sqrt(jnp.sum(diff * diff, axis=1, keepdims=True))
    pre = gs_ref[...] + gr_ref[...] + dist * wd_ref[...] + mb1_ref[...]
    h = _silu(pre)
    m = jnp.dot(h, w2t_ref[...], preferred_element_type=jnp.float32) + mb2_ref[...]
    msg_ref[...] = _silu(m)


def _edge_mlp(gs, gr, ps, pr, wd, mb1, w2t, mb2, rb):
    e, d = gs.shape
    dp = ps.shape[1]
    grid = (e // rb,)
    return pl.pallas_call(
        _edge_mlp_body,
        grid=grid,
        in_specs=[
            pl.BlockSpec((rb, d), lambda i: (i, 0)),
            pl.BlockSpec((rb, d), lambda i: (i, 0)),
            pl.BlockSpec((rb, dp), lambda i: (i, 0)),
            pl.BlockSpec((rb, dp), lambda i: (i, 0)),
            pl.BlockSpec((1, d), lambda i: (0, 0)),
            pl.BlockSpec((1, d), lambda i: (0, 0)),
            pl.BlockSpec((d, d), lambda i: (0, 0)),
            pl.BlockSpec((1, d), lambda i: (0, 0)),
        ],
        out_specs=pl.BlockSpec((rb, d), lambda i: (i, 0)),
        out_shape=jax.ShapeDtypeStruct((e, d), jnp.float32),
    )(gs, gr, ps, pr, wd, mb1, w2t, mb2)


def _make_scatter(n, d, e):
    ks = 200
    ep = e // NW
    nchunk = ep // ks
    rpt = (n // NS) // 8 * 8
    rem = n - NS * rpt
    spans = [(r0, min(ks, rpt - r0)) for r0 in range(0, rpt, ks)]
    mesh = plsc.VectorSubcoreMesh(core_axis_name="c", subcore_axis_name="s")

    @functools.partial(
        pl.kernel,
        mesh=mesh,
        out_type=jax.ShapeDtypeStruct((NC * n, d), jnp.float32),
        scratch_types=[
            pltpu.VMEM_SHARED((n, d), jnp.float32),
            pltpu.VMEM((ks, d), jnp.float32),
            pltpu.VMEM((ks,), jnp.int32),
            pltpu.SemaphoreType.DMA,
        ],
    )
    def scatter_kernel(msg_hbm, rec_hbm, zero_hbm, out_hbm, aggr_sh, mbuf,
                       ridx_v, sem):
        c = lax.axis_index("c")
        s = lax.axis_index("s")
        wid = s * NC + c
        rows0 = s * rpt
        for r0, nr in spans:
            pltpu.sync_copy(zero_hbm.at[pl.ds(0, nr)],
                            aggr_sh.at[pl.ds(rows0 + r0, nr)])

        @pl.when(s == NS - 1)
        def _zero_rem():
            pltpu.sync_copy(zero_hbm.at[pl.ds(0, rem)],
                            aggr_sh.at[pl.ds(NS * rpt, rem)])

        plsc.subcore_barrier()

        def chunk(i, carry):
            base = wid * ep + i * ks
            pltpu.sync_copy(rec_hbm.at[pl.ds(base, ks)], ridx_v)
            pltpu.sync_copy(msg_hbm.at[pl.ds(base, ks)], mbuf)
            pltpu.sync_copy(mbuf, aggr_sh.at[ridx_v], add=True)
            return carry

        lax.fori_loop(0, nchunk, chunk, 0)
        plsc.subcore_barrier()
        for r0, nr in spans:
            pltpu.sync_copy(aggr_sh.at[pl.ds(rows0 + r0, nr)],
                            mbuf.at[pl.ds(0, nr)])
            pltpu.sync_copy(mbuf.at[pl.ds(0, nr)],
                            out_hbm.at[pl.ds(c * n + rows0 + r0, nr)])

        @pl.when(s == NS - 1)
        def _write_rem():
            pltpu.sync_copy(aggr_sh.at[pl.ds(NS * rpt, rem)],
                            mbuf.at[pl.ds(0, rem)])
            pltpu.sync_copy(mbuf.at[pl.ds(0, rem)],
                            out_hbm.at[pl.ds(c * n + NS * rpt, rem)])

    return scatter_kernel


def _node_mlp_body(x_ref, p0_ref, p1_ref, wxt_ref, wat_ref, ub1_ref,
                   uw2t_ref, ub2_ref, out_ref):
    aggr = p0_ref[...] + p1_ref[...]
    pre = (jnp.dot(x_ref[...], wxt_ref[...], preferred_element_type=jnp.float32)
           + jnp.dot(aggr, wat_ref[...], preferred_element_type=jnp.float32)
           + ub1_ref[...])
    u = _silu(pre)
    out_ref[...] = (jnp.dot(u, uw2t_ref[...], preferred_element_type=jnp.float32)
                    + ub2_ref[...])


def _node_mlp(x, partials, wxt, wat, ub1, uw2t, ub2, nb):
    n, d = x.shape
    nblocks = n // nb
    grid = (nblocks,)
    return pl.pallas_call(
        _node_mlp_body,
        grid=grid,
        in_specs=[
            pl.BlockSpec((nb, d), lambda i: (i, 0)),
            pl.BlockSpec((nb, d), lambda i: (i, 0)),
            pl.BlockSpec((nb, d), lambda i, nblocks=nblocks: (i + nblocks, 0)),
            pl.BlockSpec((d, d), lambda i: (0, 0)),
            pl.BlockSpec((d, d), lambda i: (0, 0)),
            pl.BlockSpec((1, d), lambda i: (0, 0)),
            pl.BlockSpec((d, d), lambda i: (0, 0)),
            pl.BlockSpec((1, d), lambda i: (0, 0)),
        ],
        out_specs=pl.BlockSpec((nb, d), lambda i: (i, 0)),
        out_shape=jax.ShapeDtypeStruct((n, d), jnp.float32),
    )(x, partials, partials, wxt, wat, ub1, uw2t, ub2)


def kernel(x, pos, edge_index, mW1, mb1, mW2, mb2, uW1, ub1, uW2, ub2):
    n, d = x.shape
    e = edge_index.shape[1]
    assert e % (NW * K) == 0 and n % NS == 0 and n % 8 == 0

    send = edge_index[0]
    rec = edge_index[1]
    wst = mW1[:, :d].T
    wrt = mW1[:, d:2 * d].T
    wd = mW1[:, 2 * d].reshape(1, d)

    xs, xr = _precompute(x, wst, wrt, 2000)

    dp = 8
    pos_pad = jnp.zeros((n, dp), jnp.float32).at[:, :3].set(pos)
    gs, gr, ps, pr = _make_gather(n, d, e, dp)(xs, xr, send, rec, pos_pad)

    msg = _edge_mlp(gs, gr, ps, pr, wd, mb1.reshape(1, d),
                    mW2.T, mb2.reshape(1, d), 1280)

    zero = jnp.zeros((200, d), jnp.float32)
    partials = _make_scatter(n, d, e)(msg, rec, zero)

    return _node_mlp(x, partials, uW1[:, :d].T, uW1[:, d:].T,
                     ub1.reshape(1, d), uW2.T, ub2.reshape(1, d), 2000)


# speedup vs baseline: 4.0758x; 1.0852x over previous
import functools

import jax
import jax.numpy as jnp
from jax import lax
from jax.experimental import pallas as pl
from jax.experimental.pallas import tpu as pltpu
from jax.experimental.pallas import tpu_sc as plsc

NC = 2
NS = 16
NW = NC * NS
K = 400


def _sigmoid(v):
    return 1.0 / (1.0 + jnp.exp(-v))


def _silu(v):
    return v * _sigmoid(v)


def _precompute_body(x_ref, wst_ref, wrt_ref, xs_ref, xr_ref):
    xb = x_ref[...]
    xs_ref[...] = jnp.dot(xb, wst_ref[...], preferred_element_type=jnp.float32)
    xr_ref[...] = jnp.dot(xb, wrt_ref[...], preferred_element_type=jnp.float32)


def _precompute(x, wst, wrt, nb):
    n, d = x.shape
    grid = (n // nb,)
    return pl.pallas_call(
        _precompute_body,
        grid=grid,
        in_specs=[
            pl.BlockSpec((nb, d), lambda i: (i, 0)),
            pl.BlockSpec((d, d), lambda i: (0, 0)),
            pl.BlockSpec((d, d), lambda i: (0, 0)),
        ],
        out_specs=[
            pl.BlockSpec((nb, d), lambda i: (i, 0)),
            pl.BlockSpec((nb, d), lambda i: (i, 0)),
        ],
        out_shape=[
            jax.ShapeDtypeStruct((n, d), jnp.float32),
            jax.ShapeDtypeStruct((n, d), jnp.float32),
        ],
    )(x, wst, wrt)


def _make_gather_feat(n, d, e):
    ep = e // NW
    nchunk = ep // K
    mesh = plsc.VectorSubcoreMesh(core_axis_name="c", subcore_axis_name="s")

    @functools.partial(
        pl.kernel,
        mesh=mesh,
        out_type=[
            jax.ShapeDtypeStruct((e, d), jnp.float32),
            jax.ShapeDtypeStruct((e, d), jnp.float32),
        ],
        scratch_types=[
            pltpu.VMEM((K,), jnp.int32),
            pltpu.VMEM((K,), jnp.int32),
            pltpu.VMEM((K, d), jnp.float32),
            pltpu.VMEM((K, d), jnp.float32),
            pltpu.SemaphoreType.DMA,
            pltpu.SemaphoreType.DMA,
        ],
    )
    def gather_kernel(xs_hbm, xr_hbm, send_hbm, rec_hbm,
                      gs_hbm, gr_hbm,
                      sidx_v, ridx_v, sbuf, rbuf, ssem, rsem):
        c = lax.axis_index("c")
        s = lax.axis_index("s")
        wid = s * NC + c

        def chunk(i, carry):
            base = wid * ep + i * K
            pltpu.sync_copy(send_hbm.at[pl.ds(base, K)], sidx_v)
            pltpu.sync_copy(rec_hbm.at[pl.ds(base, K)], ridx_v)
            scp = pltpu.async_copy(xs_hbm.at[sidx_v], sbuf, ssem)
            rcp = pltpu.async_copy(xr_hbm.at[ridx_v], rbuf, rsem)
            scp.wait()
            pltpu.sync_copy(sbuf, gs_hbm.at[pl.ds(base, K)])
            rcp.wait()
            pltpu.sync_copy(rbuf, gr_hbm.at[pl.ds(base, K)])
            return carry

        lax.fori_loop(0, nchunk, chunk, 0)

    return gather_kernel


def _make_gather_pos(n, e, dp):
    ep = e // NW
    nchunk = ep // K
    mesh = plsc.VectorSubcoreMesh(core_axis_name="c", subcore_axis_name="s")

    @functools.partial(
        pl.kernel,
        mesh=mesh,
        out_type=[
            jax.ShapeDtypeStruct((e, dp), jnp.float32),
            jax.ShapeDtypeStruct((e, dp), jnp.float32),
        ],
        scratch_types=[
            pltpu.VMEM((K,), jnp.int32),
            pltpu.VMEM((K,), jnp.int32),
            pltpu.VMEM((K, dp), jnp.float32),
            pltpu.VMEM((K, dp), jnp.float32),
            pltpu.SemaphoreType.DMA,
            pltpu.SemaphoreType.DMA,
        ],
        compiler_params=pltpu.CompilerParams(use_tc_tiling_on_sc=False),
    )
    def gather_kernel(send_hbm, rec_hbm, pp_hbm,
                      ps_hbm, pr_hbm,
                      sidx_v, ridx_v, sbuf, rbuf, ssem, rsem):
        c = lax.axis_index("c")
        s = lax.axis_index("s")
        wid = s * NC + c

        def chunk(i, carry):
            base = wid * ep + i * K
            pltpu.sync_copy(send_hbm.at[pl.ds(base, K)], sidx_v)
            pltpu.sync_copy(rec_hbm.at[pl.ds(base, K)], ridx_v)
            scp = pltpu.async_copy(pp_hbm.at[sidx_v], sbuf, ssem)
            rcp = pltpu.async_copy(pp_hbm.at[ridx_v], rbuf, rsem)
            scp.wait()
            pltpu.sync_copy(sbuf, ps_hbm.at[pl.ds(base, K)])
            rcp.wait()
            pltpu.sync_copy(rbuf, pr_hbm.at[pl.ds(base, K)])
            return carry

        lax.fori_loop(0, nchunk, chunk, 0)

    return gather_kernel


def _edge_mlp_body(gs_ref, gr_ref, ps_ref, pr_ref, wd_ref, mb1_ref, w2t_ref,
                   mb2_ref, msg_ref):
    diff = ps_ref[...] - pr_ref[...]
    dist = jnp.sqrt(jnp.sum(diff * diff, axis=1, keepdims=True))
    pre = gs_ref[...] + gr_ref[...] + dist * wd_ref[...] + mb1_ref[...]
    h = _silu(pre)
    m = jnp.dot(h, w2t_ref[...], preferred_element_type=jnp.float32) + mb2_ref[...]
    msg_ref[...] = _silu(m)


def _edge_mlp(gs, gr, ps, pr, wd, mb1, w2t, mb2, rb):
    e, d = gs.shape
    dp = ps.shape[1]
    grid = (e // rb,)
    return pl.pallas_call(
        _edge_mlp_body,
        grid=grid,
        in_specs=[
            pl.BlockSpec((rb, d), lambda i: (i, 0)),
            pl.BlockSpec((rb, d), lambda i: (i, 0)),
            pl.BlockSpec((rb, dp), lambda i: (i, 0)),
            pl.BlockSpec((rb, dp), lambda i: (i, 0)),
            pl.BlockSpec((1, d), lambda i: (0, 0)),
            pl.BlockSpec((1, d), lambda i: (0, 0)),
            pl.BlockSpec((d, d), lambda i: (0, 0)),
            pl.BlockSpec((1, d), lambda i: (0, 0)),
        ],
        out_specs=pl.BlockSpec((rb, d), lambda i: (i, 0)),
        out_shape=jax.ShapeDtypeStruct((e, d), jnp.float32),
    )(gs, gr, ps, pr, wd, mb1, w2t, mb2)


def _make_scatter(n, d, e):
    ks = 200
    ep = e // NW
    nchunk = ep // ks
    rpt = (n // NS) // 8 * 8
    rem = n - NS * rpt
    spans = [(r0, min(ks, rpt - r0)) for r0 in range(0, rpt, ks)]
    mesh = plsc.VectorSubcoreMesh(core_axis_name="c", subcore_axis_name="s")

    @functools.partial(
        pl.kernel,
        mesh=mesh,
        out_type=jax.ShapeDtypeStruct((NC * n, d), jnp.float32),
        scratch_types=[
            pltpu.VMEM_SHARED((n, d), jnp.float32),
            pltpu.VMEM((ks, d), jnp.float32),
            pltpu.VMEM((ks,), jnp.int32),
            pltpu.SemaphoreType.DMA,
        ],
    )
    def scatter_kernel(msg_hbm, rec_hbm, zero_hbm, out_hbm, aggr_sh, mbuf,
                       ridx_v, sem):
        c = lax.axis_index("c")
        s = lax.axis_index("s")
        wid = s * NC + c
        rows0 = s * rpt
        for r0, nr in spans:
            pltpu.sync_copy(zero_hbm.at[pl.ds(0, nr)],
                            aggr_sh.at[pl.ds(rows0 + r0, nr)])

        @pl.when(s == NS - 1)
        def _zero_rem():
            pltpu.sync_copy(zero_hbm.at[pl.ds(0, rem)],
                            aggr_sh.at[pl.ds(NS * rpt, rem)])

        plsc.subcore_barrier()

        def chunk(i, carry):
            base = wid * ep + i * ks
            pltpu.sync_copy(rec_hbm.at[pl.ds(base, ks)], ridx_v)
            pltpu.sync_copy(msg_hbm.at[pl.ds(base, ks)], mbuf)
            pltpu.sync_copy(mbuf, aggr_sh.at[ridx_v], add=True)
            return carry

        lax.fori_loop(0, nchunk, chunk, 0)
        plsc.subcore_barrier()
        for r0, nr in spans:
            pltpu.sync_copy(aggr_sh.at[pl.ds(rows0 + r0, nr)],
                            mbuf.at[pl.ds(0, nr)])
            pltpu.sync_copy(mbuf.at[pl.ds(0, nr)],
                            out_hbm.at[pl.ds(c * n + rows0 + r0, nr)])

        @pl.when(s == NS - 1)
        def _write_rem():
            pltpu.sync_copy(aggr_sh.at[pl.ds(NS * rpt, rem)],
                            mbuf.at[pl.ds(0, rem)])
            pltpu.sync_copy(mbuf.at[pl.ds(0, rem)],
                            out_hbm.at[pl.ds(c * n + NS * rpt, rem)])

    return scatter_kernel


def _node_mlp_body(x_ref, p0_ref, p1_ref, wxt_ref, wat_ref, ub1_ref,
                   uw2t_ref, ub2_ref, out_ref):
    aggr = p0_ref[...] + p1_ref[...]
    pre = (jnp.dot(x_ref[...], wxt_ref[...], preferred_element_type=jnp.float32)
           + jnp.dot(aggr, wat_ref[...], preferred_element_type=jnp.float32)
           + ub1_ref[...])
    u = _silu(pre)
    out_ref[...] = (jnp.dot(u, uw2t_ref[...], preferred_element_type=jnp.float32)
                    + ub2_ref[...])


def _node_mlp(x, partials, wxt, wat, ub1, uw2t, ub2, nb):
    n, d = x.shape
    nblocks = n // nb
    grid = (nblocks,)
    return pl.pallas_call(
        _node_mlp_body,
        grid=grid,
        in_specs=[
            pl.BlockSpec((nb, d), lambda i: (i, 0)),
            pl.BlockSpec((nb, d), lambda i: (i, 0)),
            pl.BlockSpec((nb, d), lambda i, nblocks=nblocks: (i + nblocks, 0)),
            pl.BlockSpec((d, d), lambda i: (0, 0)),
            pl.BlockSpec((d, d), lambda i: (0, 0)),
            pl.BlockSpec((1, d), lambda i: (0, 0)),
            pl.BlockSpec((d, d), lambda i: (0, 0)),
            pl.BlockSpec((1, d), lambda i: (0, 0)),
        ],
        out_specs=pl.BlockSpec((nb, d), lambda i: (i, 0)),
        out_shape=jax.ShapeDtypeStruct((n, d), jnp.float32),
    )(x, partials, partials, wxt, wat, ub1, uw2t, ub2)


def kernel(x, pos, edge_index, mW1, mb1, mW2, mb2, uW1, ub1, uW2, ub2):
    n, d = x.shape
    e = edge_index.shape[1]
    assert e % (NW * K) == 0 and n % NS == 0 and n % 8 == 0

    send = edge_index[0]
    rec = edge_index[1]
    wst = mW1[:, :d].T
    wrt = mW1[:, d:2 * d].T
    wd = mW1[:, 2 * d].reshape(1, d)

    xs, xr = _precompute(x, wst, wrt, 2000)

    dp = 8
    pos_pad = jnp.zeros((n, dp), jnp.float32).at[:, :3].set(pos)
    gs, gr = _make_gather_feat(n, d, e)(xs, xr, send, rec)
    ps, pr = _make_gather_pos(n, e, dp)(send, rec, pos_pad)

    msg = _edge_mlp(gs, gr, ps, pr, wd, mb1.reshape(1, d),
                    mW2.T, mb2.reshape(1, d), 1280)

    zero = jnp.zeros((200, d), jnp.float32)
    partials = _make_scatter(n, d, e)(msg, rec, zero)

    return _node_mlp(x, partials, uW1[:, :d].T, uW1[:, d:].T,
                     ub1.reshape(1, d), uW2.T, ub2.reshape(1, d), 2000)


# speedup vs baseline: 4.8562x; 1.1915x over previous
import functools

import jax
import jax.numpy as jnp
from jax import lax
from jax.experimental import pallas as pl
from jax.experimental.pallas import tpu as pltpu
from jax.experimental.pallas import tpu_sc as plsc

NC = 2
NS = 16
NW = NC * NS
K = 400


def _sigmoid(v):
    return 1.0 / (1.0 + jnp.exp(-v))


def _silu(v):
    return v * _sigmoid(v)


def _precompute_body(x_ref, wst_ref, wrt_ref, xs_ref, xr_ref):
    xb = x_ref[...]
    xs_ref[...] = jnp.dot(xb, wst_ref[...], preferred_element_type=jnp.float32)
    xr_ref[...] = jnp.dot(xb, wrt_ref[...], preferred_element_type=jnp.float32)


def _precompute(x, wst, wrt, nb):
    n, d = x.shape
    grid = (n // nb,)
    return pl.pallas_call(
        _precompute_body,
        grid=grid,
        in_specs=[
            pl.BlockSpec((nb, d), lambda i: (i, 0)),
            pl.BlockSpec((d, d), lambda i: (0, 0)),
            pl.BlockSpec((d, d), lambda i: (0, 0)),
        ],
        out_specs=[
            pl.BlockSpec((nb, d), lambda i: (i, 0)),
            pl.BlockSpec((nb, d), lambda i: (i, 0)),
        ],
        out_shape=[
            jax.ShapeDtypeStruct((n, d), jnp.float32),
            jax.ShapeDtypeStruct((n, d), jnp.float32),
        ],
    )(x, wst, wrt)


BLK = 128
CB = 3
KG = CB * BLK


def _make_gather(n, d, e, dp):
    blocks = e // BLK
    nb = blocks // NW
    extra = blocks % NW
    nchunk = nb // CB
    assert nb % CB == 0
    ppr = BLK * dp // 128
    mesh = plsc.VectorSubcoreMesh(core_axis_name="c", subcore_axis_name="s")

    @functools.partial(
        pl.kernel,
        mesh=mesh,
        out_type=[
            jax.ShapeDtypeStruct((e, d), jnp.float32),
            jax.ShapeDtypeStruct((e, d), jnp.float32),
            jax.ShapeDtypeStruct((e, dp), jnp.float32),
            jax.ShapeDtypeStruct((e, dp), jnp.float32),
        ],
        scratch_types=[
            pltpu.VMEM((KG,), jnp.int32),
            pltpu.VMEM((KG,), jnp.int32),
            pltpu.VMEM((KG, d), jnp.float32),
            pltpu.VMEM((KG, d), jnp.float32),
            pltpu.VMEM((KG, dp), jnp.float32),
            pltpu.VMEM((KG, dp), jnp.float32),
            pltpu.SemaphoreType.DMA,
            pltpu.SemaphoreType.DMA,
            pltpu.SemaphoreType.DMA,
            pltpu.SemaphoreType.DMA,
        ],
        compiler_params=pltpu.CompilerParams(use_tc_tiling_on_sc=False),
    )
    def gather_kernel(xs_hbm, xr_hbm, send_hbm, rec_hbm, pp_hbm,
                      gs_hbm, gr_hbm, ps_hbm, pr_hbm,
                      sidx_v, ridx_v, sbuf, rbuf, psbuf, prbuf,
                      s1, s2, s3, s4):
        c = lax.axis_index("c")
        s = lax.axis_index("s")
        wid = s * NC + c
        start = (wid * nb + jnp.minimum(wid, extra)) * BLK

        def do_chunk(base, ke, kp):
            pltpu.sync_copy(send_hbm.at[pl.ds(base, ke)],
                            sidx_v.at[pl.ds(0, ke)])
            pltpu.sync_copy(rec_hbm.at[pl.ds(base, ke)],
                            ridx_v.at[pl.ds(0, ke)])
            c1 = pltpu.async_copy(xs_hbm.at[sidx_v.at[pl.ds(0, ke)]],
                                  sbuf.at[pl.ds(0, ke)], s1)
            c2 = pltpu.async_copy(xr_hbm.at[ridx_v.at[pl.ds(0, ke)]],
                                  rbuf.at[pl.ds(0, ke)], s2)
            c3 = pltpu.async_copy(pp_hbm.at[sidx_v.at[pl.ds(0, ke)]],
                                  psbuf.at[pl.ds(0, ke)], s3)
            c4 = pltpu.async_copy(pp_hbm.at[ridx_v.at[pl.ds(0, ke)]],
                                  prbuf.at[pl.ds(0, ke)], s4)
            c1.wait()
            pltpu.sync_copy(sbuf.at[pl.ds(0, ke)], gs_hbm.at[pl.ds(base, ke)])
            c2.wait()
            pltpu.sync_copy(rbuf.at[pl.ds(0, ke)], gr_hbm.at[pl.ds(base, ke)])
            c3.wait()
            pltpu.sync_copy(psbuf.at[pl.ds(0, ke)], ps_hbm.at[pl.ds(base, ke)])
            c4.wait()
            pltpu.sync_copy(prbuf.at[pl.ds(0, ke)], pr_hbm.at[pl.ds(base, ke)])

        def chunk(i, carry):
            do_chunk(start + i * KG, KG, CB * ppr)
            return carry

        lax.fori_loop(0, nchunk, chunk, 0)

        @pl.when(wid < extra)
        def _tail_block():
            do_chunk(start + nb * BLK, BLK, ppr)

    return gather_kernel


def _edge_mlp_body(gs_ref, gr_ref, ps_ref, pr_ref, sel_ref, wdblk_ref, mb1_ref,
                   w2t_ref, mb2_ref, msg_ref):
    rb = gs_ref.shape[0]
    d = gs_ref.shape[1]
    diff = ps_ref[...] - pr_ref[...]
    d2 = jnp.dot(diff * diff, sel_ref[...],
                 preferred_element_type=jnp.float32)
    distw = jnp.dot(jnp.sqrt(d2), wdblk_ref[...],
                    preferred_element_type=jnp.float32)
    pre = (gs_ref[...] + gr_ref[...] + distw.reshape(rb, d) + mb1_ref[...])
    h = _silu(pre)
    m = jnp.dot(h, w2t_ref[...], preferred_element_type=jnp.float32) + mb2_ref[...]
    msg_ref[...] = _silu(m)


def _edge_mlp(gs, gr, psp, prp, sel, wdblk, mb1, w2t, mb2, rb):
    e, d = gs.shape
    grid = (e // rb,)
    return pl.pallas_call(
        _edge_mlp_body,
        grid=grid,
        in_specs=[
            pl.BlockSpec((rb, d), lambda i: (i, 0)),
            pl.BlockSpec((rb, d), lambda i: (i, 0)),
            pl.BlockSpec((rb // 16, 128), lambda i: (i, 0)),
            pl.BlockSpec((rb // 16, 128), lambda i: (i, 0)),
            pl.BlockSpec((128, 16), lambda i: (0, 0)),
            pl.BlockSpec((16, 16 * d), lambda i: (0, 0)),
            pl.BlockSpec((1, d), lambda i: (0, 0)),
            pl.BlockSpec((d, d), lambda i: (0, 0)),
            pl.BlockSpec((1, d), lambda i: (0, 0)),
        ],
        out_specs=pl.BlockSpec((rb, d), lambda i: (i, 0)),
        out_shape=jax.ShapeDtypeStruct((e, d), jnp.float32),
    )(gs, gr, psp, prp, sel, wdblk, mb1, w2t, mb2)


def _make_scatter(n, d, e):
    ks = 200
    ep = e // NW
    nchunk = ep // ks
    rpt = (n // NS) // 8 * 8
    rem = n - NS * rpt
    spans = [(r0, min(ks, rpt - r0)) for r0 in range(0, rpt, ks)]
    mesh = plsc.VectorSubcoreMesh(core_axis_name="c", subcore_axis_name="s")

    @functools.partial(
        pl.kernel,
        mesh=mesh,
        out_type=jax.ShapeDtypeStruct((NC * n, d), jnp.float32),
        scratch_types=[
            pltpu.VMEM_SHARED((n, d), jnp.float32),
            pltpu.VMEM((ks, d), jnp.float32),
            pltpu.VMEM((ks,), jnp.int32),
            pltpu.SemaphoreType.DMA,
        ],
    )
    def scatter_kernel(msg_hbm, rec_hbm, zero_hbm, out_hbm, aggr_sh, mbuf,
                       ridx_v, sem):
        c = lax.axis_index("c")
        s = lax.axis_index("s")
        wid = s * NC + c
        rows0 = s * rpt
        for r0, nr in spans:
            pltpu.sync_copy(zero_hbm.at[pl.ds(0, nr)],
                            aggr_sh.at[pl.ds(rows0 + r0, nr)])

        @pl.when(s == NS - 1)
        def _zero_rem():
            pltpu.sync_copy(zero_hbm.at[pl.ds(0, rem)],
                            aggr_sh.at[pl.ds(NS * rpt, rem)])

        plsc.subcore_barrier()

        def chunk(i, carry):
            base = wid * ep + i * ks
            pltpu.sync_copy(rec_hbm.at[pl.ds(base, ks)], ridx_v)
            pltpu.sync_copy(msg_hbm.at[pl.ds(base, ks)], mbuf)
            pltpu.sync_copy(mbuf, aggr_sh.at[ridx_v], add=True)
            return carry

        lax.fori_loop(0, nchunk, chunk, 0)
        plsc.subcore_barrier()
        for r0, nr in spans:
            pltpu.sync_copy(aggr_sh.at[pl.ds(rows0 + r0, nr)],
                            mbuf.at[pl.ds(0, nr)])
            pltpu.sync_copy(mbuf.at[pl.ds(0, nr)],
                            out_hbm.at[pl.ds(c * n + rows0 + r0, nr)])

        @pl.when(s == NS - 1)
        def _write_rem():
            pltpu.sync_copy(aggr_sh.at[pl.ds(NS * rpt, rem)],
                            mbuf.at[pl.ds(0, rem)])
            pltpu.sync_copy(mbuf.at[pl.ds(0, rem)],
                            out_hbm.at[pl.ds(c * n + NS * rpt, rem)])

    return scatter_kernel


def _node_mlp_body(x_ref, p0_ref, p1_ref, wxt_ref, wat_ref, ub1_ref,
                   uw2t_ref, ub2_ref, out_ref):
    aggr = p0_ref[...] + p1_ref[...]
    pre = (jnp.dot(x_ref[...], wxt_ref[...], preferred_element_type=jnp.float32)
           + jnp.dot(aggr, wat_ref[...], preferred_element_type=jnp.float32)
           + ub1_ref[...])
    u = _silu(pre)
    out_ref[...] = (jnp.dot(u, uw2t_ref[...], preferred_element_type=jnp.float32)
                    + ub2_ref[...])


def _node_mlp(x, partials, wxt, wat, ub1, uw2t, ub2, nb):
    n, d = x.shape
    nblocks = n // nb
    grid = (nblocks,)
    return pl.pallas_call(
        _node_mlp_body,
        grid=grid,
        in_specs=[
            pl.BlockSpec((nb, d), lambda i: (i, 0)),
            pl.BlockSpec((nb, d), lambda i: (i, 0)),
            pl.BlockSpec((nb, d), lambda i, nblocks=nblocks: (i + nblocks, 0)),
            pl.BlockSpec((d, d), lambda i: (0, 0)),
            pl.BlockSpec((d, d), lambda i: (0, 0)),
            pl.BlockSpec((1, d), lambda i: (0, 0)),
            pl.BlockSpec((d, d), lambda i: (0, 0)),
            pl.BlockSpec((1, d), lambda i: (0, 0)),
        ],
        out_specs=pl.BlockSpec((nb, d), lambda i: (i, 0)),
        out_shape=jax.ShapeDtypeStruct((n, d), jnp.float32),
    )(x, partials, partials, wxt, wat, ub1, uw2t, ub2)


def kernel(x, pos, edge_index, mW1, mb1, mW2, mb2, uW1, ub1, uW2, ub2):
    n, d = x.shape
    e = edge_index.shape[1]
    assert e % (NW * K) == 0 and n % NS == 0 and n % 8 == 0

    send = edge_index[0]
    rec = edge_index[1]
    wst = mW1[:, :d].T
    wrt = mW1[:, d:2 * d].T
    wd = mW1[:, 2 * d].reshape(1, d)

    xs, xr = _precompute(x, wst, wrt, 2000)

    dp = 8
    pos_pad = jnp.zeros((n, dp), jnp.float32).at[:, :3].set(pos)
    gs, gr, ps, pr = _make_gather(n, d, e, dp)(xs, xr, send, rec, pos_pad)
    psp = ps.reshape(e * dp // 128, 128)
    prp = pr.reshape(e * dp // 128, 128)
    sel = jnp.repeat(jnp.eye(16, dtype=jnp.float32), dp, axis=0)
    wdblk = jnp.einsum('ij,d->ijd', jnp.eye(16, dtype=jnp.float32),
                       wd[0]).reshape(16, 16 * d)

    msg = _edge_mlp(gs, gr, psp, prp, sel, wdblk, mb1.reshape(1, d),
                    mW2.T, mb2.reshape(1, d), 1280)

    zero = jnp.zeros((200, d), jnp.float32)
    partials = _make_scatter(n, d, e)(msg, rec, zero)

    return _node_mlp(x, partials, uW1[:, :d].T, uW1[:, d:].T,
                     ub1.reshape(1, d), uW2.T, ub2.reshape(1, d), 2000)


# speedup vs baseline: 5.3285x; 1.0973x over previous
import functools

import jax
import jax.numpy as jnp
from jax import lax
from jax.experimental import pallas as pl
from jax.experimental.pallas import tpu as pltpu
from jax.experimental.pallas import tpu_sc as plsc

NC = 2
NS = 16
NW = NC * NS
K = 400


def _sigmoid(v):
    return 1.0 / (1.0 + jnp.exp(-v))


def _silu(v):
    return v * _sigmoid(v)


def _precompute_body(x_ref, wst_ref, wrt_ref, xs_ref, xr_ref):
    xb = x_ref[...]
    xs_ref[...] = jnp.dot(xb, wst_ref[...], preferred_element_type=jnp.float32)
    xr_ref[...] = jnp.dot(xb, wrt_ref[...], preferred_element_type=jnp.float32)


def _precompute(x, wst, wrt, nb):
    n, d = x.shape
    grid = (n // nb,)
    return pl.pallas_call(
        _precompute_body,
        grid=grid,
        in_specs=[
            pl.BlockSpec((nb, d), lambda i: (i, 0)),
            pl.BlockSpec((d, d), lambda i: (0, 0)),
            pl.BlockSpec((d, d), lambda i: (0, 0)),
        ],
        out_specs=[
            pl.BlockSpec((nb, d), lambda i: (i, 0)),
            pl.BlockSpec((nb, d), lambda i: (i, 0)),
        ],
        out_shape=[
            jax.ShapeDtypeStruct((n, d), jnp.float32),
            jax.ShapeDtypeStruct((n, d), jnp.float32),
        ],
    )(x, wst, wrt)


BLK = 128
CB = 3
KG = CB * BLK


def _make_gather(n, d, e, dp):
    blocks = e // BLK
    nb = blocks // NW
    extra = blocks % NW
    nchunk = nb // CB
    assert nb % CB == 0
    ppr = BLK * dp // 128
    mesh = plsc.VectorSubcoreMesh(core_axis_name="c", subcore_axis_name="s")

    @functools.partial(
        pl.kernel,
        mesh=mesh,
        out_type=[
            jax.ShapeDtypeStruct((e, d), jnp.float32),
            jax.ShapeDtypeStruct((e, dp), jnp.float32),
            jax.ShapeDtypeStruct((e, dp), jnp.float32),
        ],
        scratch_types=[
            pltpu.VMEM((KG,), jnp.int32),
            pltpu.VMEM((KG,), jnp.int32),
            pltpu.VMEM((KG, d), jnp.float32),
            pltpu.VMEM((KG, dp), jnp.float32),
            pltpu.VMEM((KG, dp), jnp.float32),
            pltpu.SemaphoreType.DMA,
            pltpu.SemaphoreType.DMA,
            pltpu.SemaphoreType.DMA,
            pltpu.SemaphoreType.DMA,
        ],
        compiler_params=pltpu.CompilerParams(use_tc_tiling_on_sc=False),
    )
    def gather_kernel(xs_hbm, xr_hbm, send_hbm, rec_hbm, pp_hbm,
                      gsum_hbm, ps_hbm, pr_hbm,
                      sidx_v, ridx_v, sbuf, psbuf, prbuf,
                      s1, s2, s3, s4):
        c = lax.axis_index("c")
        s = lax.axis_index("s")
        wid = s * NC + c
        start = (wid * nb + jnp.minimum(wid, extra)) * BLK

        def do_chunk(base, ke, kp):
            pltpu.sync_copy(send_hbm.at[pl.ds(base, ke)],
                            sidx_v.at[pl.ds(0, ke)])
            pltpu.sync_copy(rec_hbm.at[pl.ds(base, ke)],
                            ridx_v.at[pl.ds(0, ke)])
            c1 = pltpu.async_copy(xs_hbm.at[sidx_v.at[pl.ds(0, ke)]],
                                  sbuf.at[pl.ds(0, ke)], s1)
            c3 = pltpu.async_copy(pp_hbm.at[sidx_v.at[pl.ds(0, ke)]],
                                  psbuf.at[pl.ds(0, ke)], s3)
            c4 = pltpu.async_copy(pp_hbm.at[ridx_v.at[pl.ds(0, ke)]],
                                  prbuf.at[pl.ds(0, ke)], s4)
            c1.wait()
            c2 = pltpu.async_copy(xr_hbm.at[ridx_v.at[pl.ds(0, ke)]],
                                  sbuf.at[pl.ds(0, ke)], s2, add=True)
            c3.wait()
            pltpu.sync_copy(psbuf.at[pl.ds(0, ke)], ps_hbm.at[pl.ds(base, ke)])
            c4.wait()
            pltpu.sync_copy(prbuf.at[pl.ds(0, ke)], pr_hbm.at[pl.ds(base, ke)])
            c2.wait()
            pltpu.sync_copy(sbuf.at[pl.ds(0, ke)], gsum_hbm.at[pl.ds(base, ke)])

        def chunk(i, carry):
            do_chunk(start + i * KG, KG, CB * ppr)
            return carry

        lax.fori_loop(0, nchunk, chunk, 0)

        @pl.when(wid < extra)
        def _tail_block():
            do_chunk(start + nb * BLK, BLK, ppr)

    return gather_kernel


def _edge_mlp_body(gsum_ref, ps_ref, pr_ref, sel_ref, wdblk_ref, mb1_ref,
                   w2t_ref, mb2_ref, msg_ref):
    rb = gsum_ref.shape[0]
    d = gsum_ref.shape[1]
    diff = ps_ref[...] - pr_ref[...]
    d2 = jnp.dot(diff * diff, sel_ref[...],
                 preferred_element_type=jnp.float32)
    distw = jnp.dot(jnp.sqrt(d2), wdblk_ref[...],
                    preferred_element_type=jnp.float32)
    pre = (gsum_ref[...] + distw.reshape(rb, d) + mb1_ref[...])
    h = _silu(pre)
    m = jnp.dot(h, w2t_ref[...], preferred_element_type=jnp.float32) + mb2_ref[...]
    msg_ref[...] = _silu(m)


def _edge_mlp(gsum, psp, prp, sel, wdblk, mb1, w2t, mb2, rb):
    e, d = gsum.shape
    grid = (e // rb,)
    return pl.pallas_call(
        _edge_mlp_body,
        grid=grid,
        in_specs=[
            pl.BlockSpec((rb, d), lambda i: (i, 0)),
            pl.BlockSpec((rb // 16, 128), lambda i: (i, 0)),
            pl.BlockSpec((rb // 16, 128), lambda i: (i, 0)),
            pl.BlockSpec((128, 16), lambda i: (0, 0)),
            pl.BlockSpec((16, 16 * d), lambda i: (0, 0)),
            pl.BlockSpec((1, d), lambda i: (0, 0)),
            pl.BlockSpec((d, d), lambda i: (0, 0)),
            pl.BlockSpec((1, d), lambda i: (0, 0)),
        ],
        out_specs=pl.BlockSpec((rb, d), lambda i: (i, 0)),
        out_shape=jax.ShapeDtypeStruct((e, d), jnp.float32),
    )(gsum, psp, prp, sel, wdblk, mb1, w2t, mb2)


def _make_scatter(n, d, e):
    ks = 200
    ep = e // NW
    nchunk = ep // ks
    rpt = (n // NS) // 8 * 8
    rem = n - NS * rpt
    spans = [(r0, min(ks, rpt - r0)) for r0 in range(0, rpt, ks)]
    mesh = plsc.VectorSubcoreMesh(core_axis_name="c", subcore_axis_name="s")

    @functools.partial(
        pl.kernel,
        mesh=mesh,
        out_type=jax.ShapeDtypeStruct((NC * n, d), jnp.float32),
        scratch_types=[
            pltpu.VMEM_SHARED((n, d), jnp.float32),
            pltpu.VMEM((ks, d), jnp.float32),
            pltpu.VMEM((ks,), jnp.int32),
            pltpu.SemaphoreType.DMA,
        ],
    )
    def scatter_kernel(msg_hbm, rec_hbm, zero_hbm, out_hbm, aggr_sh, mbuf,
                       ridx_v, sem):
        c = lax.axis_index("c")
        s = lax.axis_index("s")
        wid = s * NC + c
        rows0 = s * rpt
        for r0, nr in spans:
            pltpu.sync_copy(zero_hbm.at[pl.ds(0, nr)],
                            aggr_sh.at[pl.ds(rows0 + r0, nr)])

        @pl.when(s == NS - 1)
        def _zero_rem():
            pltpu.sync_copy(zero_hbm.at[pl.ds(0, rem)],
                            aggr_sh.at[pl.ds(NS * rpt, rem)])

        plsc.subcore_barrier()

        def chunk(i, carry):
            base = wid * ep + i * ks
            pltpu.sync_copy(rec_hbm.at[pl.ds(base, ks)], ridx_v)
            pltpu.sync_copy(msg_hbm.at[pl.ds(base, ks)], mbuf)
            pltpu.sync_copy(mbuf, aggr_sh.at[ridx_v], add=True)
            return carry

        lax.fori_loop(0, nchunk, chunk, 0)
        plsc.subcore_barrier()
        for r0, nr in spans:
            pltpu.sync_copy(aggr_sh.at[pl.ds(rows0 + r0, nr)],
                            mbuf.at[pl.ds(0, nr)])
            pltpu.sync_copy(mbuf.at[pl.ds(0, nr)],
                            out_hbm.at[pl.ds(c * n + rows0 + r0, nr)])

        @pl.when(s == NS - 1)
        def _write_rem():
            pltpu.sync_copy(aggr_sh.at[pl.ds(NS * rpt, rem)],
                            mbuf.at[pl.ds(0, rem)])
            pltpu.sync_copy(mbuf.at[pl.ds(0, rem)],
                            out_hbm.at[pl.ds(c * n + NS * rpt, rem)])

    return scatter_kernel


def _node_mlp_body(x_ref, p0_ref, p1_ref, wxt_ref, wat_ref, ub1_ref,
                   uw2t_ref, ub2_ref, out_ref):
    aggr = p0_ref[...] + p1_ref[...]
    pre = (jnp.dot(x_ref[...], wxt_ref[...], preferred_element_type=jnp.float32)
           + jnp.dot(aggr, wat_ref[...], preferred_element_type=jnp.float32)
           + ub1_ref[...])
    u = _silu(pre)
    out_ref[...] = (jnp.dot(u, uw2t_ref[...], preferred_element_type=jnp.float32)
                    + ub2_ref[...])


def _node_mlp(x, partials, wxt, wat, ub1, uw2t, ub2, nb):
    n, d = x.shape
    nblocks = n // nb
    grid = (nblocks,)
    return pl.pallas_call(
        _node_mlp_body,
        grid=grid,
        in_specs=[
            pl.BlockSpec((nb, d), lambda i: (i, 0)),
            pl.BlockSpec((nb, d), lambda i: (i, 0)),
            pl.BlockSpec((nb, d), lambda i, nblocks=nblocks: (i + nblocks, 0)),
            pl.BlockSpec((d, d), lambda i: (0, 0)),
            pl.BlockSpec((d, d), lambda i: (0, 0)),
            pl.BlockSpec((1, d), lambda i: (0, 0)),
            pl.BlockSpec((d, d), lambda i: (0, 0)),
            pl.BlockSpec((1, d), lambda i: (0, 0)),
        ],
        out_specs=pl.BlockSpec((nb, d), lambda i: (i, 0)),
        out_shape=jax.ShapeDtypeStruct((n, d), jnp.float32),
    )(x, partials, partials, wxt, wat, ub1, uw2t, ub2)


def kernel(x, pos, edge_index, mW1, mb1, mW2, mb2, uW1, ub1, uW2, ub2):
    n, d = x.shape
    e = edge_index.shape[1]
    assert e % (NW * K) == 0 and n % NS == 0 and n % 8 == 0

    send = edge_index[0]
    rec = edge_index[1]
    wst = mW1[:, :d].T
    wrt = mW1[:, d:2 * d].T
    wd = mW1[:, 2 * d].reshape(1, d)

    xs, xr = _precompute(x, wst, wrt, 2000)

    dp = 8
    pos_pad = jnp.zeros((n, dp), jnp.float32).at[:, :3].set(pos)
    gsum, ps, pr = _make_gather(n, d, e, dp)(xs, xr, send, rec, pos_pad)
    psp = ps.reshape(e * dp // 128, 128)
    prp = pr.reshape(e * dp // 128, 128)
    sel = jnp.repeat(jnp.eye(16, dtype=jnp.float32), dp, axis=0)
    wdblk = jnp.einsum('ij,d->ijd', jnp.eye(16, dtype=jnp.float32),
                       wd[0]).reshape(16, 16 * d)

    msg = _edge_mlp(gsum, psp, prp, sel, wdblk, mb1.reshape(1, d),
                    mW2.T, mb2.reshape(1, d), 1280)

    zero = jnp.zeros((200, d), jnp.float32)
    partials = _make_scatter(n, d, e)(msg, rec, zero)

    return _node_mlp(x, partials, uW1[:, :d].T, uW1[:, d:].T,
                     ub1.reshape(1, d), uW2.T, ub2.reshape(1, d), 2000)
